# Initial kernel scaffold; baseline (speedup 1.0000x reference)
import jax
import jax.numpy as jnp
from jax.experimental import pallas as pl


def kernel(x, d, scale):
    raise NotImplementedError("write your pallas kernel here")


# speedup vs baseline: 1.0362x; 1.0362x over previous
import functools

import jax
import jax.numpy as jnp
from jax import lax
from jax.experimental import pallas as pl
from jax.experimental.pallas import tpu as pltpu
from jax.experimental.pallas import tpu_sc as plsc

N = 100000
WIDTH = 128
MAX_DEGREE = 4
L = 16
NW = 32
RPT = N // NW
CHUNK = 125
NCHUNK = RPT // CHUNK
DLEN = 3152
DPAD = 100048
GROUPS = WIDTH // L


def _sc_body(x_hbm, d_hbm, scale_hbm, out_hbm,
             scv, escv, dv, xb, ob):
    cid = lax.axis_index("c")
    sid = lax.axis_index("s")
    wid = sid * 2 + cid
    base = wid * RPT
    ab = (base // 8) * 8
    off = base - ab

    pltpu.sync_copy(d_hbm.at[pl.ds(ab, DLEN)], dv)
    pltpu.sync_copy(scale_hbm, scv)
    for i in range(MAX_DEGREE):
        for j in range(GROUPS):
            escv[i, pl.ds(j * L, L)] = jnp.exp(scv[i, pl.ds(j * L, L)])

    def chunk_body(ch, carry):
        row0 = base + ch * CHUNK
        pltpu.sync_copy(x_hbm.at[pl.ds(row0, CHUNK)], xb)

        def row_body(r, c2):
            dr = dv[pl.ds(off + ch * CHUNK + r, L)][0]
            for j in range(GROUPS):
                ob[r, pl.ds(j * L, L)] = (
                    xb[r, pl.ds(j * L, L)] * escv[dr, pl.ds(j * L, L)]
                )
            return c2

        lax.fori_loop(0, CHUNK, row_body, 0)
        pltpu.sync_copy(ob, out_hbm.at[pl.ds(row0, CHUNK)])
        return carry

    lax.fori_loop(0, NCHUNK, chunk_body, 0)


def kernel(x, d, scale):
    d32 = jnp.pad(d.astype(jnp.int32), (0, DPAD - N))
    mesh = plsc.VectorSubcoreMesh(core_axis_name="c", subcore_axis_name="s")
    f = pl.kernel(
        _sc_body,
        out_type=jax.ShapeDtypeStruct((N, WIDTH), jnp.float32),
        mesh=mesh,
        scratch_types=[
            pltpu.VMEM((MAX_DEGREE, WIDTH), jnp.float32),
            pltpu.VMEM((MAX_DEGREE, WIDTH), jnp.float32),
            pltpu.VMEM((DLEN,), jnp.int32),
            pltpu.VMEM((CHUNK, WIDTH), jnp.float32),
            pltpu.VMEM((CHUNK, WIDTH), jnp.float32),
        ],
        compiler_params=pltpu.CompilerParams(use_tc_tiling_on_sc=False),
    )
    return f(x, d32, scale)


# speedup vs baseline: 3.7882x; 3.6558x over previous
import jax
import jax.numpy as jnp
from jax import lax
from jax.experimental import pallas as pl
from jax.experimental.pallas import tpu as pltpu
from jax.experimental.pallas import tpu_sc as plsc

N = 100000
WIDTH = 128
MAX_DEGREE = 4
L = 16
NW = 32
RPT = N // NW
CHUNK = 125
CPAD = 128
NCHUNK = RPT // CHUNK
DLEN = 3152
DPAD = 100048
GROUPS = WIDTH // L
RGRP = CPAD // L


def _sc_body(x_hbm, d_hbm, scale_hbm, out_hbm,
             scv, dv, xb0, xb1, ob0, ob1,
             in_sem0, in_sem1, out_sem0, out_sem1):
    cid = lax.axis_index("c")
    sid = lax.axis_index("s")
    wid = sid * 2 + cid
    base = wid * RPT
    ab = (base // 8) * 8
    off = base - ab

    pltpu.sync_copy(d_hbm.at[pl.ds(ab, DLEN)], dv)
    pltpu.sync_copy(scale_hbm, scv)
    esc = [[jnp.exp(scv[i, pl.ds(j * L, L)]) for j in range(GROUPS)]
           for i in range(MAX_DEGREE)]

    def in_copy(buf, sem, ch):
        return pltpu.make_async_copy(
            x_hbm.at[pl.ds(base + ch * CHUNK, CHUNK)],
            buf.at[pl.ds(0, CHUNK)], sem)

    def out_copy(buf, sem, ch):
        return pltpu.make_async_copy(
            buf.at[pl.ds(0, CHUNK)],
            out_hbm.at[pl.ds(base + ch * CHUNK, CHUNK)], sem)

    def compute(xbuf, obuf, ch):
        dbase = off + ch * CHUNK

        def grp(g, carry):
            drv = dv[pl.ds(dbase + g * L, L)]
            for k in range(L):
                dr = drv[k]
                b0 = dr == 0
                b1 = dr == 1
                b2 = dr == 2
                r = g * L + k
                for j in range(GROUPS):
                    m = jnp.where(b0, esc[0][j],
                                  jnp.where(b1, esc[1][j],
                                            jnp.where(b2, esc[2][j],
                                                      esc[3][j])))
                    obuf[r, pl.ds(j * L, L)] = xbuf[r, pl.ds(j * L, L)] * m
            return carry

        lax.fori_loop(0, RGRP, grp, 0)

    in_copy(xb0, in_sem0, 0).start()

    def pair(i, carry):
        a = 2 * i
        in_copy(xb1, in_sem1, a + 1).start()
        in_copy(xb0, in_sem0, a).wait()

        @pl.when(i > 0)
        def _():
            out_copy(ob0, out_sem0, a - 2).wait()

        compute(xb0, ob0, a)
        out_copy(ob0, out_sem0, a).start()

        @pl.when(a + 2 < NCHUNK)
        def _():
            in_copy(xb0, in_sem0, a + 2).start()

        in_copy(xb1, in_sem1, a + 1).wait()

        @pl.when(i > 0)
        def _():
            out_copy(ob1, out_sem1, a - 1).wait()

        compute(xb1, ob1, a + 1)
        out_copy(ob1, out_sem1, a + 1).start()
        return carry

    lax.fori_loop(0, NCHUNK // 2, pair, 0)

    last = NCHUNK - 1
    in_copy(xb0, in_sem0, last).wait()
    out_copy(ob0, out_sem0, last - 2).wait()
    compute(xb0, ob0, last)
    out_copy(ob0, out_sem0, last).start()
    out_copy(ob0, out_sem0, last).wait()
    out_copy(ob1, out_sem1, last - 1).wait()


def kernel(x, d, scale):
    d32 = jnp.pad(d.astype(jnp.int32), (0, DPAD - N))
    mesh = plsc.VectorSubcoreMesh(core_axis_name="c", subcore_axis_name="s")
    f = pl.kernel(
        _sc_body,
        out_type=jax.ShapeDtypeStruct((N, WIDTH), jnp.float32),
        mesh=mesh,
        scratch_types=[
            pltpu.VMEM((MAX_DEGREE, WIDTH), jnp.float32),
            pltpu.VMEM((DLEN,), jnp.int32),
            pltpu.VMEM((CPAD, WIDTH), jnp.float32),
            pltpu.VMEM((CPAD, WIDTH), jnp.float32),
            pltpu.VMEM((CPAD, WIDTH), jnp.float32),
            pltpu.VMEM((CPAD, WIDTH), jnp.float32),
            pltpu.SemaphoreType.DMA,
            pltpu.SemaphoreType.DMA,
            pltpu.SemaphoreType.DMA,
            pltpu.SemaphoreType.DMA,
        ],
        compiler_params=pltpu.CompilerParams(use_tc_tiling_on_sc=False),
    )
    return f(x, d32, scale)


# speedup vs baseline: 4.0338x; 1.0648x over previous
import jax
import jax.numpy as jnp
from jax import lax
from jax.experimental import pallas as pl
from jax.experimental.pallas import tpu as pltpu
from jax.experimental.pallas import tpu_sc as plsc

N = 100000
WIDTH = 128
MAX_DEGREE = 4
L = 16
NW = 32
RPT = N // NW
CHUNK = 125
CPAD = 128
NCHUNK = RPT // CHUNK
DLEN = 3152
DPAD = 100048
GROUPS = WIDTH // L
RGRP = CPAD // L


NBUF = 3


def _sc_body(x_hbm, d_hbm, scale_hbm, out_hbm,
             scv, dv, xb0, xb1, xb2, ob0, ob1, ob2,
             in_sem0, in_sem1, in_sem2, out_sem0, out_sem1, out_sem2):
    cid = lax.axis_index("c")
    sid = lax.axis_index("s")
    wid = sid * 2 + cid
    base = wid * RPT
    ab = (base // 8) * 8
    off = base - ab

    pltpu.sync_copy(d_hbm.at[pl.ds(ab, DLEN)], dv)
    pltpu.sync_copy(scale_hbm, scv)
    esc = [[jnp.exp(scv[i, pl.ds(j * L, L)]) for j in range(GROUPS)]
           for i in range(MAX_DEGREE)]

    def in_copy(buf, sem, ch):
        return pltpu.make_async_copy(
            x_hbm.at[pl.ds(base + ch * CHUNK, CHUNK)],
            buf.at[pl.ds(0, CHUNK)], sem)

    def out_copy(buf, sem, ch):
        return pltpu.make_async_copy(
            buf.at[pl.ds(0, CHUNK)],
            out_hbm.at[pl.ds(base + ch * CHUNK, CHUNK)], sem)

    def compute(xbuf, obuf, ch):
        dbase = off + ch * CHUNK

        def grp(g, carry):
            drv = dv[pl.ds(dbase + g * L, L)]
            for k in range(L):
                dr = drv[k]
                b0 = dr == 0
                b1 = dr == 1
                b2 = dr == 2
                r = g * L + k
                for j in range(GROUPS):
                    m = jnp.where(b0, esc[0][j],
                                  jnp.where(b1, esc[1][j],
                                            jnp.where(b2, esc[2][j],
                                                      esc[3][j])))
                    obuf[r, pl.ds(j * L, L)] = xbuf[r, pl.ds(j * L, L)] * m
            return carry

        lax.fori_loop(0, RGRP, grp, 0)

    xbs = [xb0, xb1, xb2]
    obs = [ob0, ob1, ob2]
    in_sems = [in_sem0, in_sem1, in_sem2]
    out_sems = [out_sem0, out_sem1, out_sem2]

    for b in range(NBUF):
        in_copy(xbs[b], in_sems[b], b).start()

    def round_body(i, carry):
        for b in range(NBUF):
            ch = NBUF * i + b
            in_copy(xbs[b], in_sems[b], ch).wait()

            @pl.when(i > 0)
            def _():
                out_copy(obs[b], out_sems[b], ch - NBUF).wait()

            compute(xbs[b], obs[b], ch)
            out_copy(obs[b], out_sems[b], ch).start()

            @pl.when(ch + NBUF < NCHUNK)
            def _():
                in_copy(xbs[b], in_sems[b], ch + NBUF).start()
        return carry

    lax.fori_loop(0, NCHUNK // NBUF, round_body, 0)

    last = NCHUNK - 1
    in_copy(xbs[0], in_sems[0], last).wait()
    out_copy(obs[0], out_sems[0], last - NBUF).wait()
    compute(xbs[0], obs[0], last)
    out_copy(obs[0], out_sems[0], last).start()
    out_copy(obs[0], out_sems[0], last).wait()
    for b in range(1, NBUF):
        out_copy(obs[b], out_sems[b], last - NBUF + b).wait()


def kernel(x, d, scale):
    d32 = jnp.pad(d.astype(jnp.int32), (0, DPAD - N))
    mesh = plsc.VectorSubcoreMesh(core_axis_name="c", subcore_axis_name="s")
    f = pl.kernel(
        _sc_body,
        out_type=jax.ShapeDtypeStruct((N, WIDTH), jnp.float32),
        mesh=mesh,
        scratch_types=[
            pltpu.VMEM((MAX_DEGREE, WIDTH), jnp.float32),
            pltpu.VMEM((DLEN,), jnp.int32),
            pltpu.VMEM((CPAD, WIDTH), jnp.float32),
            pltpu.VMEM((CPAD, WIDTH), jnp.float32),
            pltpu.VMEM((CPAD, WIDTH), jnp.float32),
            pltpu.VMEM((CPAD, WIDTH), jnp.float32),
            pltpu.VMEM((CPAD, WIDTH), jnp.float32),
            pltpu.VMEM((CPAD, WIDTH), jnp.float32),
            pltpu.SemaphoreType.DMA,
            pltpu.SemaphoreType.DMA,
            pltpu.SemaphoreType.DMA,
            pltpu.SemaphoreType.DMA,
            pltpu.SemaphoreType.DMA,
            pltpu.SemaphoreType.DMA,
        ],
        compiler_params=pltpu.CompilerParams(use_tc_tiling_on_sc=False),
    )
    return f(x, d32, scale)
